# baseline (speedup 1.0000x reference)
import functools

import jax
import jax.numpy as jnp
import numpy as np
from jax import lax
from jax.experimental import pallas as pl
from jax.experimental.pallas import tpu as pltpu
from jax.experimental.pallas import tpu_sc as plsc

B = 128
V = 100000
W = 128
TOPK = 50
TOP_P = 0.9
ROWS_PER = 4
CHUNK = 20000
NCH = V // CHUNK
NVREG = CHUNK // 16
PL = 1312
EARLY = 64

_U32 = np.uint32
_SIGN = _U32(0x80000000)


def _key_from_val(v):
    bu = lax.bitcast_convert_type(v, jnp.uint32)
    return jnp.where(bu >= _SIGN, ~bu, bu | _SIGN)


def _val_from_key(k):
    bu = jnp.where(k >= _SIGN, k & ~_SIGN, ~k)
    return lax.bitcast_convert_type(bu, jnp.float32)


def _threefry_uniform(k1v, k2v, flat_u32):
    rots = ((13, 15, 26, 6), (17, 29, 16, 24))
    ks0, ks1 = k1v, k2v
    ks2 = ks0 ^ ks1 ^ _U32(0x1BD11BDA)
    ks = (ks0, ks1, ks2)
    x0 = jnp.zeros_like(flat_u32) + ks0
    x1 = flat_u32 + ks1
    for i in range(5):
        for r in rots[i % 2]:
            x0 = x0 + x1
            x1 = (x1 << _U32(r)) | (x1 >> _U32(32 - r))
            x1 = x1 ^ x0
        x0 = x0 + ks[(i + 1) % 3]
        x1 = x1 + ks[(i + 2) % 3] + _U32(i + 1)
    bits = x0 ^ x1
    fb = lax.bitcast_convert_type((bits >> _U32(9)) | _U32(0x3F800000),
                                  jnp.float32)
    f = fb - jnp.float32(1.0)
    tiny = jnp.float32(np.finfo(np.float32).tiny)
    return jnp.maximum(tiny, f * (jnp.float32(1.0) - tiny) + tiny)


def _popcnt(m):
    return jnp.sum(m.astype(jnp.int32))


def _k1_body(x_hbm, kv_hbm, ov_hbm, oi_hbm, ou_hbm,
             buf, poolk, pooli, sv, si, su, keybuf, bfly):
    wid = lax.axis_index("s") * 2 + lax.axis_index("c")
    lanes = lax.broadcasted_iota(jnp.int32, (16,), 0)
    neginf = jnp.full((16,), -jnp.inf, jnp.float32)
    zero16 = jnp.zeros((16,), jnp.int32)
    trash = jnp.full((16,), PL * 16, jnp.int32) + lanes
    pltpu.sync_copy(kv_hbm, keybuf)
    k1v = keybuf[pl.ds(0, 16)]
    k2v = keybuf[pl.ds(16, 16)]

    def splat_total(x):
        acc = x
        for k in (1, 2, 4, 8):
            bfly[...] = acc
            acc = acc + plsc.load_gather(bfly, [lanes ^ k])
        return acc

    def pool_count_ge(mid, cntv, nslot):
        def cb(s, acc):
            ku = lax.bitcast_convert_type(poolk[pl.ds(s * 16, 16)],
                                          jnp.uint32)
            m = (ku >= mid) & (s < cntv)
            return acc + m.astype(jnp.int32)

        return lax.fori_loop(0, nslot, cb, zero16)

    def kth_key_vec(cntv, nslot, niter):
        lo = jnp.zeros((16,), jnp.uint32)
        hi = lo + _U32(0xFFFFFFFE)

        def bs(_, lohi):
            lo, hi = lohi
            mid = lo + ((hi - lo + _U32(1)) >> _U32(1))
            tot = splat_total(pool_count_ge(mid, cntv, nslot))
            big = tot >= TOPK
            return (jnp.where(big, mid, lo),
                    jnp.where(big, hi, mid - _U32(1)))

        lo, _ = lax.fori_loop(0, niter, bs, (lo, hi))
        return lo

    def shrink(cntv, niter):
        nslot = jnp.max(cntv)
        tk = kth_key_vec(cntv, nslot, niter)

        def comp(s, newcntv):
            ki = poolk[pl.ds(s * 16, 16)]
            ii = pooli[pl.ds(s * 16, 16)]
            ku = lax.bitcast_convert_type(ki, jnp.uint32)
            m = (ku >= tk) & (s < cntv)
            tgt = jnp.where(m, newcntv * 16 + lanes, trash)
            plsc.store_scatter(poolk, [tgt], ki)
            plsc.store_scatter(pooli, [tgt], ii)
            return newcntv + m.astype(jnp.int32)

        newcntv = lax.fori_loop(0, nslot, comp, zero16)
        return newcntv, tk

    def scan_vregs(lo_vreg, hi_vreg, cbase, carry):
        def step(i, ct):
            cntv, tval = ct
            v = buf[pl.ds(i * 16, 16)]
            m = v >= tval
            mi = m.astype(jnp.int32)
            cc = jnp.minimum(cntv, PL - 1)
            tgt = jnp.where(m, cc * 16 + lanes, trash)
            plsc.store_scatter(
                poolk, [tgt],
                lax.bitcast_convert_type(_key_from_val(v), jnp.int32))
            plsc.store_scatter(pooli, [tgt],
                               cbase + i * 16 + lanes)
            return cntv + mi, tval

        return lax.fori_loop(lo_vreg, hi_vreg, step, carry)

    def row_fn(r, _):
        row = wid * ROWS_PER + r
        rbase = row * V
        cntv = zero16
        tval = neginf
        for c in range(NCH):
            pltpu.sync_copy(x_hbm.at[pl.ds(rbase + c * CHUNK, CHUNK)], buf)
            if c == 0:
                cntv, tval = scan_vregs(0, EARLY, c * CHUNK, (cntv, tval))
                cntv, tk = shrink(cntv, 24)
                tval = _val_from_key(tk)
                cntv, tval = scan_vregs(EARLY, NVREG, c * CHUNK,
                                        (cntv, tval))
            else:
                cntv, tval = scan_vregs(0, NVREG, c * CHUNK, (cntv, tval))
            cntv, tk = shrink(cntv, 24)
            tval = _val_from_key(tk)

        nslot = jnp.max(cntv)
        tk = kth_key_vec(cntv, nslot, 32)
        for j in range(W // 16):
            sv[pl.ds(j * 16, 16)] = neginf
            si[pl.ds(j * 16, 16)] = zero16
        oc = zero16
        strash = jnp.full((16,), W, jnp.int32)

        def fcomp(s, oc):
            ki = poolk[pl.ds(s * 16, 16)]
            ii = pooli[pl.ds(s * 16, 16)]
            ku = lax.bitcast_convert_type(ki, jnp.uint32)
            m = (ku >= tk) & (s < cntv)
            mi = m.astype(jnp.int32)
            pos = oc + plsc.cumsum(mi) - 1
            tgt = jnp.where(m & (pos < W), pos, strash)
            plsc.store_scatter(sv, [tgt], _val_from_key(ku))
            plsc.store_scatter(si, [tgt], ii)
            return oc + splat_total(mi)

        lax.fori_loop(0, nslot, fcomp, oc)
        for j in range(W // 16):
            flat = (rbase + si[pl.ds(j * 16, 16)]).astype(jnp.uint32)
            su[pl.ds(j * 16, 16)] = _threefry_uniform(k1v, k2v, flat)
        obase = row * W
        pltpu.sync_copy(sv.at[pl.ds(0, W)], ov_hbm.at[pl.ds(obase, W)])
        pltpu.sync_copy(si.at[pl.ds(0, W)], oi_hbm.at[pl.ds(obase, W)])
        pltpu.sync_copy(su, ou_hbm.at[pl.ds(obase, W)])
        return 0

    lax.fori_loop(0, ROWS_PER, row_fn, 0)


def _k1_call(x_flat, kv):
    mesh = plsc.VectorSubcoreMesh(core_axis_name="c", subcore_axis_name="s")
    return pl.kernel(
        _k1_body,
        mesh=mesh,
        out_type=[jax.ShapeDtypeStruct((B * W,), jnp.float32),
                  jax.ShapeDtypeStruct((B * W,), jnp.int32),
                  jax.ShapeDtypeStruct((B * W,), jnp.float32)],
        scratch_types=[pltpu.VMEM((CHUNK,), jnp.float32),
                       pltpu.VMEM((PL * 16 + 16,), jnp.int32),
                       pltpu.VMEM((PL * 16 + 16,), jnp.int32),
                       pltpu.VMEM((W + 16,), jnp.float32),
                       pltpu.VMEM((W + 16,), jnp.int32),
                       pltpu.VMEM((W,), jnp.float32),
                       pltpu.VMEM((32,), jnp.uint32),
                       pltpu.VMEM((16,), jnp.int32)],
        compiler_params=pltpu.CompilerParams(needs_layout_passes=False),
    )(x_flat, kv)


def _k2_body(v_ref, i_ref, g_ref, s_ref, p_ref):
    v = v_ref[...]
    idx = i_ref[...]
    g = g_ref[...]
    valid = v > jnp.float32(-1e38)
    rowmax = jnp.max(v, axis=1, keepdims=True)
    p = jnp.where(valid, jnp.exp(v - rowmax), jnp.float32(0.0))
    z = jnp.sum(p, axis=1, keepdims=True)
    cdfnum = jnp.zeros_like(v)
    for m in range(W):
        vm = v[:, m:m + 1]
        im = idx[:, m:m + 1]
        pm = p[:, m:m + 1]
        lex_le = (vm < v) | ((vm == v) & (im <= idx))
        cdfnum = cdfnum + jnp.where(lex_le, pm, jnp.float32(0.0))
    maxidx = jnp.max(jnp.where(v == rowmax, idx, jnp.int32(-1)),
                     axis=1, keepdims=True)
    is_top = (v == rowmax) & (idx == maxidx)
    remove = (cdfnum <= jnp.float32(1.0 - TOP_P) * z) & jnp.logical_not(is_top)
    kept = valid & jnp.logical_not(remove)
    z2 = jnp.sum(jnp.where(kept, p, jnp.float32(0.0)), axis=1, keepdims=True)
    probs = p / z2
    score = jnp.where(kept, v + g, -jnp.inf)
    smax = jnp.max(score, axis=1, keepdims=True)
    winner = score == smax
    sample = jnp.min(jnp.where(winner, idx, jnp.int32(2**31 - 1)),
                     axis=1, keepdims=True)
    sprob = jnp.sum(jnp.where(winner & (idx == sample), probs,
                              jnp.float32(0.0)), axis=1, keepdims=True)
    s_ref[...] = jnp.broadcast_to(sample, s_ref.shape)
    p_ref[...] = jnp.broadcast_to(sprob, p_ref.shape)


def _k2_call(cv, ci, g):
    return pl.pallas_call(
        _k2_body,
        out_shape=[jax.ShapeDtypeStruct((B, W), jnp.int32),
                   jax.ShapeDtypeStruct((B, W), jnp.float32)],
    )(cv, ci, g)


def kernel(logits, top_k):
    b, l, v = logits.shape
    x_flat = logits.reshape(b * v)
    kd = jax.random.key_data(
        jax.random.fold_in(jax.random.key(0), 1)).astype(jnp.uint32)
    kv = jnp.concatenate([jnp.broadcast_to(kd[0], (16,)),
                          jnp.broadcast_to(kd[1], (16,))])
    cvf, cif, cuf = _k1_call(x_flat, kv)
    cv = cvf.reshape(B, W)
    ci = cif.reshape(B, W)
    cu = cuf.reshape(B, W)
    g = -jnp.log(-jnp.log(cu))
    s, p = _k2_call(cv, ci, g)
    samples = s[:, :1].reshape(b, l, 1)
    sample_probs = p[:, :1].reshape(b, l, 1)
    return samples, sample_probs


# speedup vs baseline: 1.3431x; 1.3431x over previous
import functools

import jax
import jax.numpy as jnp
import numpy as np
from jax import lax
from jax.experimental import pallas as pl
from jax.experimental.pallas import tpu as pltpu
from jax.experimental.pallas import tpu_sc as plsc

B = 128
V = 100000
W = 128
TOPK = 50
TOP_P = 0.9
ROWS_PER = 4
CHUNK = 20000
NCH = V // CHUNK
NVREG = CHUNK // 16
PL = 1312
GV = 10
NG = NVREG // GV
EARLYG = 5
WLCAP = 128

_U32 = np.uint32
_SIGN = _U32(0x80000000)


def _key_from_val(v):
    bu = lax.bitcast_convert_type(v, jnp.uint32)
    return jnp.where(bu >= _SIGN, ~bu, bu | _SIGN)


def _val_from_key(k):
    bu = jnp.where(k >= _SIGN, k & ~_SIGN, ~k)
    return lax.bitcast_convert_type(bu, jnp.float32)


def _threefry_uniform(k1v, k2v, flat_u32):
    rots = ((13, 15, 26, 6), (17, 29, 16, 24))
    ks0, ks1 = k1v, k2v
    ks2 = ks0 ^ ks1 ^ _U32(0x1BD11BDA)
    ks = (ks0, ks1, ks2)
    x0 = jnp.zeros_like(flat_u32) + ks0
    x1 = flat_u32 + ks1
    for i in range(5):
        for r in rots[i % 2]:
            x0 = x0 + x1
            x1 = (x1 << _U32(r)) | (x1 >> _U32(32 - r))
            x1 = x1 ^ x0
        x0 = x0 + ks[(i + 1) % 3]
        x1 = x1 + ks[(i + 2) % 3] + _U32(i + 1)
    bits = x0 ^ x1
    fb = lax.bitcast_convert_type((bits >> _U32(9)) | _U32(0x3F800000),
                                  jnp.float32)
    f = fb - jnp.float32(1.0)
    tiny = jnp.float32(np.finfo(np.float32).tiny)
    return jnp.maximum(tiny, f * (jnp.float32(1.0) - tiny) + tiny)


def _popcnt(m):
    return jnp.sum(m.astype(jnp.int32))


def _k1_body(x_hbm, kv_hbm, ov_hbm, oi_hbm, ou_hbm,
             buf, poolk, pooli, sv, si, su, keybuf, bfly, wl):
    wid = lax.axis_index("s") * 2 + lax.axis_index("c")
    lanes = lax.broadcasted_iota(jnp.int32, (16,), 0)
    neginf = jnp.full((16,), -jnp.inf, jnp.float32)
    zero16 = jnp.zeros((16,), jnp.int32)
    trash = jnp.full((16,), PL * 16, jnp.int32) + lanes
    pltpu.sync_copy(kv_hbm, keybuf)
    k1v = keybuf[pl.ds(0, 16)]
    k2v = keybuf[pl.ds(16, 16)]

    def splat_total(x):
        acc = x
        for k in (1, 2, 4, 8):
            bfly[...] = acc
            acc = acc + plsc.load_gather(bfly, [lanes ^ k])
        return acc

    def pool_count_ge(mid, cntv, nslot):
        def cb(s, acc):
            ku = lax.bitcast_convert_type(poolk[pl.ds(s * 16, 16)],
                                          jnp.uint32)
            m = (ku >= mid) & (s < cntv)
            return acc + m.astype(jnp.int32)

        return lax.fori_loop(0, nslot, cb, zero16)

    def kth_key_vec(cntv, nslot, niter):
        lo = jnp.zeros((16,), jnp.uint32)
        hi = lo + _U32(0xFFFFFFFE)

        def bs(_, lohi):
            lo, hi = lohi
            mid = lo + ((hi - lo + _U32(1)) >> _U32(1))
            tot = splat_total(pool_count_ge(mid, cntv, nslot))
            big = tot >= TOPK
            return (jnp.where(big, mid, lo),
                    jnp.where(big, hi, mid - _U32(1)))

        lo, _ = lax.fori_loop(0, niter, bs, (lo, hi))
        return lo

    def shrink(cntv, niter):
        nslot = jnp.max(cntv)
        tk = kth_key_vec(cntv, nslot, niter)

        def comp(s, newcntv):
            ki = poolk[pl.ds(s * 16, 16)]
            ii = pooli[pl.ds(s * 16, 16)]
            ku = lax.bitcast_convert_type(ki, jnp.uint32)
            m = (ku >= tk) & (s < cntv)
            tgt = jnp.where(m, newcntv * 16 + lanes, trash)
            plsc.store_scatter(poolk, [tgt], ki)
            plsc.store_scatter(pooli, [tgt], ii)
            return newcntv + m.astype(jnp.int32)

        newcntv = lax.fori_loop(0, nslot, comp, zero16)
        return newcntv, tk

    def scan_vregs(lo_vreg, hi_vreg, cbase, carry):
        def step(i, ct):
            cntv, tval = ct
            v = buf[pl.ds(i * 16, 16)]
            m = v >= tval
            mi = m.astype(jnp.int32)
            cc = jnp.minimum(cntv, PL - 1)
            tgt = jnp.where(m, cc * 16 + lanes, trash)
            plsc.store_scatter(
                poolk, [tgt],
                lax.bitcast_convert_type(_key_from_val(v), jnp.int32))
            plsc.store_scatter(pooli, [tgt],
                               cbase + i * 16 + lanes)
            return cntv + mi, tval

        return lax.fori_loop(lo_vreg, hi_vreg, step, carry)

    wtrash = jnp.full((16,), WLCAP * 16, jnp.int32) + lanes

    def chunk_worklist(cbase, g0, cntv, tval):
        def p1(g, wc):
            gm = buf[pl.ds(g * (GV * 16), 16)]
            for j in range(1, GV):
                gm = jnp.maximum(gm, buf[pl.ds(g * (GV * 16) + j * 16, 16)])
            m = gm >= tval
            tgt = jnp.where(m, wc * 16 + lanes, wtrash)
            plsc.store_scatter(wl, [tgt], zero16 + g)
            return wc + m.astype(jnp.int32)

        wcntv = lax.fori_loop(g0, NG, p1, zero16)
        nw = jnp.max(wcntv)

        def p2(s, cntv):
            gvec = plsc.load_gather(wl, [s * 16 + lanes])
            gvec = jnp.minimum(jnp.maximum(gvec, 0), NG - 1)
            act = s < wcntv
            for j in range(GV):
                eidx = gvec * (GV * 16) + j * 16 + lanes
                v = plsc.load_gather(buf, [eidx])
                m = (v >= tval) & act
                cc = jnp.minimum(cntv, PL - 1)
                tgt = jnp.where(m, cc * 16 + lanes, trash)
                plsc.store_scatter(
                    poolk, [tgt],
                    lax.bitcast_convert_type(_key_from_val(v), jnp.int32))
                plsc.store_scatter(pooli, [tgt], cbase + eidx)
                cntv = cntv + m.astype(jnp.int32)
            return cntv

        return lax.fori_loop(0, nw, p2, cntv)

    def row_fn(r, _):
        row = wid * ROWS_PER + r
        rbase = row * V
        cntv = zero16
        tval = neginf
        for c in range(NCH):
            pltpu.sync_copy(x_hbm.at[pl.ds(rbase + c * CHUNK, CHUNK)], buf)
            if c == 0:
                cntv, tval = scan_vregs(0, EARLYG * GV, 0, (cntv, tval))
                cntv, tk = shrink(cntv, 24)
                tval = _val_from_key(tk)
                cntv = chunk_worklist(0, EARLYG, cntv, tval)
            else:
                cntv = chunk_worklist(c * CHUNK, 0, cntv, tval)
            cntv, tk = shrink(cntv, 24)
            tval = _val_from_key(tk)

        nslot = jnp.max(cntv)
        tk = kth_key_vec(cntv, nslot, 32)
        for j in range(W // 16):
            sv[pl.ds(j * 16, 16)] = neginf
            si[pl.ds(j * 16, 16)] = zero16
        oc = zero16
        strash = jnp.full((16,), W, jnp.int32)

        def fcomp(s, oc):
            ki = poolk[pl.ds(s * 16, 16)]
            ii = pooli[pl.ds(s * 16, 16)]
            ku = lax.bitcast_convert_type(ki, jnp.uint32)
            m = (ku >= tk) & (s < cntv)
            mi = m.astype(jnp.int32)
            pos = oc + plsc.cumsum(mi) - 1
            tgt = jnp.where(m & (pos < W), pos, strash)
            plsc.store_scatter(sv, [tgt], _val_from_key(ku))
            plsc.store_scatter(si, [tgt], ii)
            return oc + splat_total(mi)

        lax.fori_loop(0, nslot, fcomp, oc)
        for j in range(W // 16):
            flat = (rbase + si[pl.ds(j * 16, 16)]).astype(jnp.uint32)
            su[pl.ds(j * 16, 16)] = _threefry_uniform(k1v, k2v, flat)
        obase = row * W
        pltpu.sync_copy(sv.at[pl.ds(0, W)], ov_hbm.at[pl.ds(obase, W)])
        pltpu.sync_copy(si.at[pl.ds(0, W)], oi_hbm.at[pl.ds(obase, W)])
        pltpu.sync_copy(su, ou_hbm.at[pl.ds(obase, W)])
        return 0

    lax.fori_loop(0, ROWS_PER, row_fn, 0)


def _k1_call(x_flat, kv):
    mesh = plsc.VectorSubcoreMesh(core_axis_name="c", subcore_axis_name="s")
    return pl.kernel(
        _k1_body,
        mesh=mesh,
        out_type=[jax.ShapeDtypeStruct((B * W,), jnp.float32),
                  jax.ShapeDtypeStruct((B * W,), jnp.int32),
                  jax.ShapeDtypeStruct((B * W,), jnp.float32)],
        scratch_types=[pltpu.VMEM((CHUNK,), jnp.float32),
                       pltpu.VMEM((PL * 16 + 16,), jnp.int32),
                       pltpu.VMEM((PL * 16 + 16,), jnp.int32),
                       pltpu.VMEM((W + 16,), jnp.float32),
                       pltpu.VMEM((W + 16,), jnp.int32),
                       pltpu.VMEM((W,), jnp.float32),
                       pltpu.VMEM((32,), jnp.uint32),
                       pltpu.VMEM((16,), jnp.int32),
                       pltpu.VMEM((WLCAP * 16 + 16,), jnp.int32)],
        compiler_params=pltpu.CompilerParams(needs_layout_passes=False),
    )(x_flat, kv)


def _k2_body(v_ref, i_ref, g_ref, s_ref, p_ref):
    v = v_ref[...]
    idx = i_ref[...]
    g = g_ref[...]
    valid = v > jnp.float32(-1e38)
    rowmax = jnp.max(v, axis=1, keepdims=True)
    p = jnp.where(valid, jnp.exp(v - rowmax), jnp.float32(0.0))
    z = jnp.sum(p, axis=1, keepdims=True)
    cdfnum = jnp.zeros_like(v)
    for m in range(W):
        vm = v[:, m:m + 1]
        im = idx[:, m:m + 1]
        pm = p[:, m:m + 1]
        lex_le = (vm < v) | ((vm == v) & (im <= idx))
        cdfnum = cdfnum + jnp.where(lex_le, pm, jnp.float32(0.0))
    maxidx = jnp.max(jnp.where(v == rowmax, idx, jnp.int32(-1)),
                     axis=1, keepdims=True)
    is_top = (v == rowmax) & (idx == maxidx)
    remove = (cdfnum <= jnp.float32(1.0 - TOP_P) * z) & jnp.logical_not(is_top)
    kept = valid & jnp.logical_not(remove)
    z2 = jnp.sum(jnp.where(kept, p, jnp.float32(0.0)), axis=1, keepdims=True)
    probs = p / z2
    score = jnp.where(kept, v + g, -jnp.inf)
    smax = jnp.max(score, axis=1, keepdims=True)
    winner = score == smax
    sample = jnp.min(jnp.where(winner, idx, jnp.int32(2**31 - 1)),
                     axis=1, keepdims=True)
    sprob = jnp.sum(jnp.where(winner & (idx == sample), probs,
                              jnp.float32(0.0)), axis=1, keepdims=True)
    s_ref[...] = jnp.broadcast_to(sample, s_ref.shape)
    p_ref[...] = jnp.broadcast_to(sprob, p_ref.shape)


def _k2_call(cv, ci, g):
    return pl.pallas_call(
        _k2_body,
        out_shape=[jax.ShapeDtypeStruct((B, W), jnp.int32),
                   jax.ShapeDtypeStruct((B, W), jnp.float32)],
    )(cv, ci, g)


def kernel(logits, top_k):
    b, l, v = logits.shape
    x_flat = logits.reshape(b * v)
    kd = jax.random.key_data(
        jax.random.fold_in(jax.random.key(0), 1)).astype(jnp.uint32)
    kv = jnp.concatenate([jnp.broadcast_to(kd[0], (16,)),
                          jnp.broadcast_to(kd[1], (16,))])
    cvf, cif, cuf = _k1_call(x_flat, kv)
    cv = cvf.reshape(B, W)
    ci = cif.reshape(B, W)
    cu = cuf.reshape(B, W)
    g = -jnp.log(-jnp.log(cu))
    s, p = _k2_call(cv, ci, g)
    samples = s[:, :1].reshape(b, l, 1)
    sample_probs = p[:, :1].reshape(b, l, 1)
    return samples, sample_probs


# speedup vs baseline: 1.4188x; 1.0564x over previous
import functools

import jax
import jax.numpy as jnp
import numpy as np
from jax import lax
from jax.experimental import pallas as pl
from jax.experimental.pallas import tpu as pltpu
from jax.experimental.pallas import tpu_sc as plsc

B = 128
V = 100000
W = 128
TOPK = 50
TOP_P = 0.9
ROWS_PER = 4
CHUNK = 20000
NCH = V // CHUNK
NVREG = CHUNK // 16
PL = 1312
GV = 10
NG = NVREG // GV
EARLYG = 5
WLCAP = 128

_U32 = np.uint32
_SIGN = _U32(0x80000000)


def _key_from_val(v):
    bu = lax.bitcast_convert_type(v, jnp.uint32)
    return jnp.where(bu >= _SIGN, ~bu, bu | _SIGN)


def _val_from_key(k):
    bu = jnp.where(k >= _SIGN, k & ~_SIGN, ~k)
    return lax.bitcast_convert_type(bu, jnp.float32)


def _threefry_uniform(k1v, k2v, flat_u32):
    rots = ((13, 15, 26, 6), (17, 29, 16, 24))
    ks0, ks1 = k1v, k2v
    ks2 = ks0 ^ ks1 ^ _U32(0x1BD11BDA)
    ks = (ks0, ks1, ks2)
    x0 = jnp.zeros_like(flat_u32) + ks0
    x1 = flat_u32 + ks1
    for i in range(5):
        for r in rots[i % 2]:
            x0 = x0 + x1
            x1 = (x1 << _U32(r)) | (x1 >> _U32(32 - r))
            x1 = x1 ^ x0
        x0 = x0 + ks[(i + 1) % 3]
        x1 = x1 + ks[(i + 2) % 3] + _U32(i + 1)
    bits = x0 ^ x1
    fb = lax.bitcast_convert_type((bits >> _U32(9)) | _U32(0x3F800000),
                                  jnp.float32)
    f = fb - jnp.float32(1.0)
    tiny = jnp.float32(np.finfo(np.float32).tiny)
    return jnp.maximum(tiny, f * (jnp.float32(1.0) - tiny) + tiny)


def _popcnt(m):
    return jnp.sum(m.astype(jnp.int32))


def _k1_body(x_hbm, kv_hbm, ov_hbm, oi_hbm, ou_hbm,
             buf, poolk, pooli, sv, si, su, keybuf, bfly, wl,
             sem0, sem1):
    sems = (sem0, sem1)
    wid = lax.axis_index("s") * 2 + lax.axis_index("c")
    lanes = lax.broadcasted_iota(jnp.int32, (16,), 0)
    neginf = jnp.full((16,), -jnp.inf, jnp.float32)
    zero16 = jnp.zeros((16,), jnp.int32)
    trash = jnp.full((16,), PL * 16, jnp.int32) + lanes
    pltpu.sync_copy(kv_hbm, keybuf)
    k1v = keybuf[pl.ds(0, 16)]
    k2v = keybuf[pl.ds(16, 16)]

    def splat_total(x):
        acc = x
        for k in (1, 2, 4, 8):
            bfly[...] = acc
            acc = acc + plsc.load_gather(bfly, [lanes ^ k])
        return acc

    def pool_count_ge(mid, cntv, nslot):
        def cb(s, acc):
            ku = lax.bitcast_convert_type(poolk[pl.ds(s * 16, 16)],
                                          jnp.uint32)
            m = (ku >= mid) & (s < cntv)
            return acc + m.astype(jnp.int32)

        return lax.fori_loop(0, nslot, cb, zero16)

    def kth_key_vec(cntv, nslot, niter):
        lo = jnp.zeros((16,), jnp.uint32)
        hi = lo + _U32(0xFFFFFFFE)

        def bs(_, lohi):
            lo, hi = lohi
            mid = lo + ((hi - lo + _U32(1)) >> _U32(1))
            tot = splat_total(pool_count_ge(mid, cntv, nslot))
            big = tot >= TOPK
            return (jnp.where(big, mid, lo),
                    jnp.where(big, hi, mid - _U32(1)))

        lo, _ = lax.fori_loop(0, niter, bs, (lo, hi))
        return lo

    def shrink(cntv, niter):
        nslot = jnp.max(cntv)
        tk = kth_key_vec(cntv, nslot, niter)

        def comp(s, newcntv):
            ki = poolk[pl.ds(s * 16, 16)]
            ii = pooli[pl.ds(s * 16, 16)]
            ku = lax.bitcast_convert_type(ki, jnp.uint32)
            m = (ku >= tk) & (s < cntv)
            tgt = jnp.where(m, newcntv * 16 + lanes, trash)
            plsc.store_scatter(poolk, [tgt], ki)
            plsc.store_scatter(pooli, [tgt], ii)
            return newcntv + m.astype(jnp.int32)

        newcntv = lax.fori_loop(0, nslot, comp, zero16)
        return newcntv, tk

    def scan_vregs(bufref, lo_vreg, hi_vreg, cbase, carry):
        def step(i, ct):
            cntv, tval = ct
            v = bufref[pl.ds(i * 16, 16)]
            m = v >= tval
            mi = m.astype(jnp.int32)
            cc = jnp.minimum(cntv, PL - 1)
            tgt = jnp.where(m, cc * 16 + lanes, trash)
            plsc.store_scatter(
                poolk, [tgt],
                lax.bitcast_convert_type(_key_from_val(v), jnp.int32))
            plsc.store_scatter(pooli, [tgt],
                               cbase + i * 16 + lanes)
            return cntv + mi, tval

        return lax.fori_loop(lo_vreg, hi_vreg, step, carry)

    wtrash = jnp.full((16,), WLCAP * 16, jnp.int32) + lanes

    def chunk_worklist(bufref, cbase, g0, cntv, tval):
        def p1(g, wc):
            gm = bufref[pl.ds(g * (GV * 16), 16)]
            for j in range(1, GV):
                gm = jnp.maximum(gm,
                                 bufref[pl.ds(g * (GV * 16) + j * 16, 16)])
            m = gm >= tval
            tgt = jnp.where(m, wc * 16 + lanes, wtrash)
            plsc.store_scatter(wl, [tgt], zero16 + g)
            return wc + m.astype(jnp.int32)

        wcntv = lax.fori_loop(g0, NG, p1, zero16)
        nw = jnp.max(wcntv)

        def p2(s, cntv):
            gvec = plsc.load_gather(wl, [s * 16 + lanes])
            gvec = jnp.minimum(jnp.maximum(gvec, 0), NG - 1)
            act = s < wcntv
            for j in range(GV):
                eidx = gvec * (GV * 16) + j * 16 + lanes
                v = plsc.load_gather(bufref, [eidx])
                m = (v >= tval) & act
                cc = jnp.minimum(cntv, PL - 1)
                tgt = jnp.where(m, cc * 16 + lanes, trash)
                plsc.store_scatter(
                    poolk, [tgt],
                    lax.bitcast_convert_type(_key_from_val(v), jnp.int32))
                plsc.store_scatter(pooli, [tgt], cbase + eidx)
                cntv = cntv + m.astype(jnp.int32)
            return cntv

        return lax.fori_loop(0, nw, p2, cntv)

    def row_fn(r, _):
        row = wid * ROWS_PER + r
        rbase = row * V
        cntv = zero16
        tval = neginf
        cps = [None, None]
        cps[0] = pltpu.async_copy(x_hbm.at[pl.ds(rbase, CHUNK)],
                                  buf.at[pl.ds(0, CHUNK)], sems[0])
        for c in range(NCH):
            par = c % 2
            if c + 1 < NCH:
                nxt = (c + 1) % 2
                cps[nxt] = pltpu.async_copy(
                    x_hbm.at[pl.ds(rbase + (c + 1) * CHUNK, CHUNK)],
                    buf.at[pl.ds(nxt * CHUNK, CHUNK)], sems[nxt])
            cps[par].wait()
            bufref = buf.at[pl.ds(par * CHUNK, CHUNK)]
            if c == 0:
                cntv, tval = scan_vregs(bufref, 0, EARLYG * GV, 0,
                                        (cntv, tval))
                cntv, tk = shrink(cntv, 24)
                tval = _val_from_key(tk)
                cntv = chunk_worklist(bufref, 0, EARLYG, cntv, tval)
            else:
                cntv = chunk_worklist(bufref, c * CHUNK, 0, cntv, tval)
            cntv, tk = shrink(cntv, 24)
            tval = _val_from_key(tk)

        nslot = jnp.max(cntv)
        tk = kth_key_vec(cntv, nslot, 32)
        for j in range(W // 16):
            sv[pl.ds(j * 16, 16)] = neginf
            si[pl.ds(j * 16, 16)] = zero16
        oc = zero16
        strash = jnp.full((16,), W, jnp.int32)

        def fcomp(s, oc):
            ki = poolk[pl.ds(s * 16, 16)]
            ii = pooli[pl.ds(s * 16, 16)]
            ku = lax.bitcast_convert_type(ki, jnp.uint32)
            m = (ku >= tk) & (s < cntv)
            mi = m.astype(jnp.int32)
            pos = oc + plsc.cumsum(mi) - 1
            tgt = jnp.where(m & (pos < W), pos, strash)
            plsc.store_scatter(sv, [tgt], _val_from_key(ku))
            plsc.store_scatter(si, [tgt], ii)
            return oc + splat_total(mi)

        lax.fori_loop(0, nslot, fcomp, oc)
        for j in range(W // 16):
            flat = (rbase + si[pl.ds(j * 16, 16)]).astype(jnp.uint32)
            su[pl.ds(j * 16, 16)] = _threefry_uniform(k1v, k2v, flat)
        obase = row * W
        pltpu.sync_copy(sv.at[pl.ds(0, W)], ov_hbm.at[pl.ds(obase, W)])
        pltpu.sync_copy(si.at[pl.ds(0, W)], oi_hbm.at[pl.ds(obase, W)])
        pltpu.sync_copy(su, ou_hbm.at[pl.ds(obase, W)])
        return 0

    lax.fori_loop(0, ROWS_PER, row_fn, 0)


def _k1_call(x_flat, kv):
    mesh = plsc.VectorSubcoreMesh(core_axis_name="c", subcore_axis_name="s")
    return pl.kernel(
        _k1_body,
        mesh=mesh,
        out_type=[jax.ShapeDtypeStruct((B * W,), jnp.float32),
                  jax.ShapeDtypeStruct((B * W,), jnp.int32),
                  jax.ShapeDtypeStruct((B * W,), jnp.float32)],
        scratch_types=[pltpu.VMEM((2 * CHUNK,), jnp.float32),
                       pltpu.VMEM((PL * 16 + 16,), jnp.int32),
                       pltpu.VMEM((PL * 16 + 16,), jnp.int32),
                       pltpu.VMEM((W + 16,), jnp.float32),
                       pltpu.VMEM((W + 16,), jnp.int32),
                       pltpu.VMEM((W,), jnp.float32),
                       pltpu.VMEM((32,), jnp.uint32),
                       pltpu.VMEM((16,), jnp.int32),
                       pltpu.VMEM((WLCAP * 16 + 16,), jnp.int32),
                       pltpu.SemaphoreType.DMA,
                       pltpu.SemaphoreType.DMA],
        compiler_params=pltpu.CompilerParams(needs_layout_passes=False),
    )(x_flat, kv)


def _k2_body(v_ref, i_ref, g_ref, s_ref, p_ref):
    v = v_ref[...]
    idx = i_ref[...]
    g = g_ref[...]
    valid = v > jnp.float32(-1e38)
    rowmax = jnp.max(v, axis=1, keepdims=True)
    p = jnp.where(valid, jnp.exp(v - rowmax), jnp.float32(0.0))
    z = jnp.sum(p, axis=1, keepdims=True)
    cdfnum = jnp.zeros_like(v)
    for m in range(W):
        vm = v[:, m:m + 1]
        im = idx[:, m:m + 1]
        pm = p[:, m:m + 1]
        lex_le = (vm < v) | ((vm == v) & (im <= idx))
        cdfnum = cdfnum + jnp.where(lex_le, pm, jnp.float32(0.0))
    maxidx = jnp.max(jnp.where(v == rowmax, idx, jnp.int32(-1)),
                     axis=1, keepdims=True)
    is_top = (v == rowmax) & (idx == maxidx)
    remove = (cdfnum <= jnp.float32(1.0 - TOP_P) * z) & jnp.logical_not(is_top)
    kept = valid & jnp.logical_not(remove)
    z2 = jnp.sum(jnp.where(kept, p, jnp.float32(0.0)), axis=1, keepdims=True)
    probs = p / z2
    score = jnp.where(kept, v + g, -jnp.inf)
    smax = jnp.max(score, axis=1, keepdims=True)
    winner = score == smax
    sample = jnp.min(jnp.where(winner, idx, jnp.int32(2**31 - 1)),
                     axis=1, keepdims=True)
    sprob = jnp.sum(jnp.where(winner & (idx == sample), probs,
                              jnp.float32(0.0)), axis=1, keepdims=True)
    s_ref[...] = jnp.broadcast_to(sample, s_ref.shape)
    p_ref[...] = jnp.broadcast_to(sprob, p_ref.shape)


def _k2_call(cv, ci, g):
    return pl.pallas_call(
        _k2_body,
        out_shape=[jax.ShapeDtypeStruct((B, W), jnp.int32),
                   jax.ShapeDtypeStruct((B, W), jnp.float32)],
    )(cv, ci, g)


def kernel(logits, top_k):
    b, l, v = logits.shape
    x_flat = logits.reshape(b * v)
    kd = jax.random.key_data(
        jax.random.fold_in(jax.random.key(0), 1)).astype(jnp.uint32)
    kv = jnp.concatenate([jnp.broadcast_to(kd[0], (16,)),
                          jnp.broadcast_to(kd[1], (16,))])
    cvf, cif, cuf = _k1_call(x_flat, kv)
    cv = cvf.reshape(B, W)
    ci = cif.reshape(B, W)
    cu = cuf.reshape(B, W)
    g = -jnp.log(-jnp.log(cu))
    s, p = _k2_call(cv, ci, g)
    samples = s[:, :1].reshape(b, l, 1)
    sample_probs = p[:, :1].reshape(b, l, 1)
    return samples, sample_probs
